# Initial kernel scaffold; baseline (speedup 1.0000x reference)
import jax
import jax.numpy as jnp
from jax.experimental import pallas as pl


def kernel(x):
    raise NotImplementedError("write your pallas kernel here")


# speedup vs baseline: 1.9857x; 1.9857x over previous
import jax
import jax.numpy as jnp
import numpy as np
from jax.experimental import pallas as pl

_ANCHORS = np.array(
    [[16, 8], [23, 103], [28, 23], [56, 47], [96, 123], [157, 248]],
    dtype=np.float32,
)
_NUM_CLASSES = 80
_IMG_DIM = 512.0
_NA = 6
_ATTRS = 5 + _NUM_CLASSES
_NG = 64
_NPOS = _NG * _NG
_STRIDE = _IMG_DIM / _NG

_PCHUNK = 1024
_NPC = _NPOS // _PCHUNK


def _decode_kernel(x_ref, o_ref):
    p = pl.program_id(1)
    t = x_ref[0]
    iota = jax.lax.broadcasted_iota(jnp.int32, (1, _PCHUNK), 1)
    gx = (iota % _NG).astype(jnp.float32)
    gy = (p * (_PCHUNK // _NG) + iota // _NG).astype(jnp.float32)
    for a in range(_NA):
        base = a * _ATTRS
        blk = t[base:base + _ATTRS, :]
        r0 = (jax.nn.sigmoid(blk[0:1, :]) + gx) * _STRIDE
        r1 = (jax.nn.sigmoid(blk[1:2, :]) + gy) * _STRIDE
        r2 = jnp.exp(blk[2:3, :]) * _ANCHORS[a, 0]
        r3 = jnp.exp(blk[3:4, :]) * _ANCHORS[a, 1]
        r4 = jax.nn.sigmoid(blk[4:5, :])
        full = jnp.concatenate([r0, r1, r2, r3, r4, blk[5:, :]], axis=0)
        o_ref[0, a, 0] = full.T


def kernel(x):
    nB = x.shape[0]
    x4 = x.reshape(nB, _NA * _ATTRS, _NPOS)
    out = pl.pallas_call(
        _decode_kernel,
        grid=(nB, _NPC),
        in_specs=[
            pl.BlockSpec((1, _NA * _ATTRS, _PCHUNK), lambda b, p: (b, 0, p)),
        ],
        out_specs=pl.BlockSpec(
            (1, _NA, 1, _PCHUNK, _ATTRS), lambda b, p: (b, 0, p, 0, 0)
        ),
        out_shape=jax.ShapeDtypeStruct(
            (nB, _NA, _NPC, _PCHUNK, _ATTRS), jnp.float32
        ),
    )(x4)
    return out.reshape(nB, _NA * _NPOS, _ATTRS)


# speedup vs baseline: 2.0532x; 1.0340x over previous
import jax
import jax.numpy as jnp
import numpy as np
from jax.experimental import pallas as pl
from jax.experimental.pallas import tpu as pltpu

_ANCHORS = np.array(
    [[16, 8], [23, 103], [28, 23], [56, 47], [96, 123], [157, 248]],
    dtype=np.float32,
)
_NUM_CLASSES = 80
_IMG_DIM = 512.0
_NA = 6
_ATTRS = 5 + _NUM_CLASSES
_NG = 64
_NPOS = _NG * _NG
_STRIDE = _IMG_DIM / _NG

_PCHUNK = 4096
_NPC = _NPOS // _PCHUNK


def _decode_kernel(x_ref, o_ref):
    p = pl.program_id(1)
    t = x_ref[0]
    iota = jax.lax.broadcasted_iota(jnp.int32, (1, _PCHUNK), 1)
    gx = (iota % _NG).astype(jnp.float32)
    gy = (p * (_PCHUNK // _NG) + iota // _NG).astype(jnp.float32)
    for a in range(_NA):
        base = a * _ATTRS
        blk = t[base:base + _ATTRS, :]
        r0 = (jax.nn.sigmoid(blk[0:1, :]) + gx) * _STRIDE
        r1 = (jax.nn.sigmoid(blk[1:2, :]) + gy) * _STRIDE
        r2 = jnp.exp(blk[2:3, :]) * _ANCHORS[a, 0]
        r3 = jnp.exp(blk[3:4, :]) * _ANCHORS[a, 1]
        r4 = jax.nn.sigmoid(blk[4:5, :])
        full = jnp.concatenate([r0, r1, r2, r3, r4, blk[5:, :]], axis=0)
        o_ref[0, a, 0] = full.T


def kernel(x):
    nB = x.shape[0]
    x4 = x.reshape(nB, _NA * _ATTRS, _NPOS)
    out = pl.pallas_call(
        _decode_kernel,
        grid=(nB, _NPC),
        in_specs=[
            pl.BlockSpec((1, _NA * _ATTRS, _PCHUNK), lambda b, p: (b, 0, p)),
        ],
        out_specs=pl.BlockSpec(
            (1, _NA, 1, _PCHUNK, _ATTRS), lambda b, p: (b, 0, p, 0, 0)
        ),
        out_shape=jax.ShapeDtypeStruct(
            (nB, _NA, _NPC, _PCHUNK, _ATTRS), jnp.float32
        ),
        compiler_params=pltpu.CompilerParams(
            dimension_semantics=("parallel", "parallel"),
        ),
    )(x4)
    return out.reshape(nB, _NA * _NPOS, _ATTRS)
